# baseline (speedup 1.0000x reference)
import functools

import jax
import jax.numpy as jnp
from jax import lax
from jax.experimental import pallas as pl
from jax.experimental.pallas import tpu as pltpu
from jax.experimental.pallas import tpu_sc as plsc

BATCH = 16384
SEQ = 200
VOCAB = 1000000
NC = 2
NS = 16
NW = NC * NS
RPW = BATCH // NW
RCHUNK = 32
CTOK = RCHUNK * SEQ
NSTEP = RPW // RCHUNK
LANES = 16
ROW_OFFS = tuple(range(0, SEQ - LANES + 1, LANES)) + (SEQ - LANES,)
EXP_OFFS = tuple(range(0, SEQ - LANES + 1, LANES))
TAIL = SEQ - LANES
STAGE_TILES = 8
STAGE_W = VOCAB // STAGE_TILES
STAGE_CHUNK = 5000
STAGE_ROUNDS = STAGE_W // STAGE_CHUNK


def _gather_kernel(idx_hbm, table_hbm, out_hbm, tab_s,
                   idx2d_0, idx2d_1, val2d_0, val2d_1,
                   idx1d_0, idx1d_1, val1d_0, val1d_1,
                   isem0, isem1, gsem0, gsem1, ssem0, ssem1):
    idx2d = (idx2d_0, idx2d_1)
    val2d = (val2d_0, val2d_1)
    idx1d = (idx1d_0, idx1d_1)
    val1d = (val1d_0, val1d_1)
    isem = (isem0, isem1)
    gsem = (gsem0, gsem1)
    ssem = (ssem0, ssem1)
    sid = lax.axis_index("s")
    wid = sid * NC + lax.axis_index("c")
    rbase = wid * RPW

    @pl.when(sid < STAGE_TILES)
    def _stage():
        tbase = sid * STAGE_W
        ld = {}
        ld[0] = pltpu.async_copy(
            table_hbm.at[pl.ds(tbase, STAGE_CHUNK)],
            val1d_0.at[pl.ds(0, STAGE_CHUNK)], gsem0)
        for j in range(STAGE_ROUNDS):
            if j + 1 < STAGE_ROUNDS:
                ld[j + 1] = pltpu.async_copy(
                    table_hbm.at[pl.ds(tbase + (j + 1) * STAGE_CHUNK,
                                       STAGE_CHUNK)],
                    val1d[(j + 1) % 2].at[pl.ds(0, STAGE_CHUNK)],
                    gsem[(j + 1) % 2])
            ld[j].wait()
            pltpu.sync_copy(val1d[j % 2].at[pl.ds(0, STAGE_CHUNK)],
                            tab_s.at[pl.ds(tbase + j * STAGE_CHUNK,
                                           STAGE_CHUNK)])

    plsc.subcore_barrier()


    def idx_copy(i):
        return pltpu.async_copy(
            idx_hbm.at[pl.ds(rbase + i * RCHUNK, RCHUNK), :], idx2d[i % 2],
            isem[i % 2])

    def gather(i):
        return pltpu.async_copy(
            tab_s.at[idx1d[i % 2]], val1d[i % 2], gsem[i % 2])

    def store(i):
        return pltpu.async_copy(
            val2d[i % 2], out_hbm.at[pl.ds(rbase + i * RCHUNK, RCHUNK), :],
            ssem[i % 2])

    UNROLL = 4

    def compact(i):
        p = i % 2

        def body(r4, carry):
            for k in range(UNROLL):
                r = r4 * UNROLL + k
                for off in ROW_OFFS:
                    idx1d[p][pl.ds(r * SEQ + off, LANES)] = (
                        idx2d[p][r, pl.ds(off, LANES)])
            return carry

        lax.fori_loop(0, RCHUNK // UNROLL, body, 0)

    tail_cols = lax.iota(jnp.int32, LANES) + TAIL
    tail_mask = tail_cols >= (TAIL + 8)

    def expand(i):
        p = i % 2

        def body(r4, carry):
            for k in range(UNROLL):
                r = r4 * UNROLL + k
                for off in EXP_OFFS:
                    val2d[p][r, pl.ds(off, LANES)] = (
                        val1d[p][pl.ds(r * SEQ + off, LANES)])
                tail = val1d[p][pl.ds(r * SEQ + TAIL, LANES)]
                rows = jnp.full((LANES,), r, jnp.int32)
                plsc.store_scatter(val2d[p], [rows, tail_cols], tail,
                                   mask=tail_mask)
            return carry

        lax.fori_loop(0, RCHUNK // UNROLL, body, 0)

    ic, gc, st = {}, {}, {}
    ic[0] = idx_copy(0)
    ic[0].wait()
    compact(0)
    gc[0] = gather(0)
    ic[1] = idx_copy(1)
    for i in range(NSTEP):
        if i + 1 < NSTEP:
            ic[i + 1].wait()
            compact(i + 1)
            gc[i + 1] = gather(i + 1)
            if i + 2 < NSTEP:
                ic[i + 2] = idx_copy(i + 2)
        gc[i].wait()
        if i >= 2:
            st[i - 2].wait()
        expand(i)
        st[i] = store(i)
    st[NSTEP - 2].wait()
    st[NSTEP - 1].wait()


def kernel(tokens, table):
    idx = tokens.astype(jnp.int32)
    mesh = plsc.VectorSubcoreMesh(core_axis_name="c", subcore_axis_name="s")
    run = functools.partial(
        pl.kernel,
        mesh=mesh,
        compiler_params=pltpu.CompilerParams(needs_layout_passes=False),
        out_type=jax.ShapeDtypeStruct((BATCH, SEQ), jnp.float32),
        scratch_types=[
            pltpu.VMEM_SHARED((VOCAB,), jnp.float32),
            pltpu.VMEM((RCHUNK, SEQ), jnp.int32),
            pltpu.VMEM((RCHUNK, SEQ), jnp.int32),
            pltpu.VMEM((RCHUNK, SEQ), jnp.float32),
            pltpu.VMEM((RCHUNK, SEQ), jnp.float32),
            pltpu.VMEM((CTOK,), jnp.int32),
            pltpu.VMEM((CTOK,), jnp.int32),
            pltpu.VMEM((CTOK,), jnp.float32),
            pltpu.VMEM((CTOK,), jnp.float32),
            pltpu.SemaphoreType.DMA,
            pltpu.SemaphoreType.DMA,
            pltpu.SemaphoreType.DMA,
            pltpu.SemaphoreType.DMA,
            pltpu.SemaphoreType.DMA,
            pltpu.SemaphoreType.DMA,
        ],
    )(_gather_kernel)
    return run(idx, table)


# speedup vs baseline: 1.0779x; 1.0779x over previous
import functools

import jax
import jax.numpy as jnp
from jax import lax
from jax.experimental import pallas as pl
from jax.experimental.pallas import tpu as pltpu
from jax.experimental.pallas import tpu_sc as plsc

BATCH = 16384
SEQ = 200
VOCAB = 1000000
NC = 2
NS = 16
NW = NC * NS
RPW = BATCH // NW
RCHUNK = 32
CTOK = RCHUNK * SEQ
NSTEP = RPW // RCHUNK
LANES = 16
ROW_OFFS = tuple(range(0, SEQ - LANES + 1, LANES)) + (SEQ - LANES,)
EXP_OFFS = tuple(range(0, SEQ - LANES + 1, LANES))
TAIL = SEQ - LANES
STAGE_A = 62496
STAGE_B = 62504
STAGE_CHUNK_A = 5208
STAGE_CHUNK_B = 4808


def _gather_kernel(idx_hbm, table_hbm, out_hbm, tab_s,
                   idx2d_0, idx2d_1, val2d_0, val2d_1,
                   idx1d_0, idx1d_1, val1d_0, val1d_1,
                   isem0, isem1, gsem0, gsem1, ssem0, ssem1):
    idx2d = (idx2d_0, idx2d_1)
    val2d = (val2d_0, val2d_1)
    idx1d = (idx1d_0, idx1d_1)
    val1d = (val1d_0, val1d_1)
    isem = (isem0, isem1)
    gsem = (gsem0, gsem1)
    ssem = (ssem0, ssem1)
    sid = lax.axis_index("s")
    wid = sid * NC + lax.axis_index("c")
    rbase = wid * RPW

    def idx_copy(i):
        return pltpu.async_copy(
            idx_hbm.at[pl.ds(rbase + i * RCHUNK, RCHUNK), :], idx2d[i % 2],
            isem[i % 2])

    ic, gc, st = {}, {}, {}
    ic[0] = idx_copy(0)
    ic[1] = idx_copy(1)

    def _stage_loop(tbase, chunk, rounds):
        ld = {}
        ld[0] = pltpu.async_copy(
            table_hbm.at[pl.ds(tbase, chunk)],
            val1d_0.at[pl.ds(0, chunk)], gsem0)
        for j in range(rounds):
            if j + 1 < rounds:
                ld[j + 1] = pltpu.async_copy(
                    table_hbm.at[pl.ds(tbase + (j + 1) * chunk, chunk)],
                    val1d[(j + 1) % 2].at[pl.ds(0, chunk)],
                    gsem[(j + 1) % 2])
            ld[j].wait()
            pltpu.sync_copy(val1d[j % 2].at[pl.ds(0, chunk)],
                            tab_s.at[pl.ds(tbase + j * chunk, chunk)])

    pair_base = (sid // 2) * (STAGE_A + STAGE_B)

    @pl.when(sid % 2 == 0)
    def _stage_even():
        _stage_loop(pair_base, STAGE_CHUNK_A, STAGE_A // STAGE_CHUNK_A)

    @pl.when(sid % 2 == 1)
    def _stage_odd():
        _stage_loop(pair_base + STAGE_A, STAGE_CHUNK_B,
                    STAGE_B // STAGE_CHUNK_B)

    plsc.subcore_barrier()

    def gather(i):
        return pltpu.async_copy(
            tab_s.at[idx1d[i % 2]], val1d[i % 2], gsem[i % 2])

    def store(i):
        return pltpu.async_copy(
            val2d[i % 2], out_hbm.at[pl.ds(rbase + i * RCHUNK, RCHUNK), :],
            ssem[i % 2])

    def compact(i):
        p = i % 2

        def body(r, carry):
            for off in ROW_OFFS:
                idx1d[p][pl.ds(r * SEQ + off, LANES)] = (
                    idx2d[p][r, pl.ds(off, LANES)])
            return carry

        lax.fori_loop(0, RCHUNK, body, 0)

    tail_cols = lax.iota(jnp.int32, LANES) + TAIL
    tail_mask = tail_cols >= (TAIL + 8)

    def expand(i):
        p = i % 2

        def body(r, carry):
            for off in EXP_OFFS:
                val2d[p][r, pl.ds(off, LANES)] = (
                    val1d[p][pl.ds(r * SEQ + off, LANES)])
            tail = val1d[p][pl.ds(r * SEQ + TAIL, LANES)]
            rows = jnp.full((LANES,), r, jnp.int32)
            plsc.store_scatter(val2d[p], [rows, tail_cols], tail,
                               mask=tail_mask)
            return carry

        lax.fori_loop(0, RCHUNK, body, 0)

    ic[0].wait()
    compact(0)
    gc[0] = gather(0)
    for i in range(NSTEP):
        if i + 1 < NSTEP:
            ic[i + 1].wait()
            compact(i + 1)
            gc[i + 1] = gather(i + 1)
            if i + 2 < NSTEP:
                ic[i + 2] = idx_copy(i + 2)
        gc[i].wait()
        if i >= 2:
            st[i - 2].wait()
        expand(i)
        st[i] = store(i)
    st[NSTEP - 2].wait()
    st[NSTEP - 1].wait()


def kernel(tokens, table):
    idx = tokens.astype(jnp.int32)
    mesh = plsc.VectorSubcoreMesh(core_axis_name="c", subcore_axis_name="s")
    run = functools.partial(
        pl.kernel,
        mesh=mesh,
        compiler_params=pltpu.CompilerParams(needs_layout_passes=False),
        out_type=jax.ShapeDtypeStruct((BATCH, SEQ), jnp.float32),
        scratch_types=[
            pltpu.VMEM_SHARED((VOCAB,), jnp.float32),
            pltpu.VMEM((RCHUNK, SEQ), jnp.int32),
            pltpu.VMEM((RCHUNK, SEQ), jnp.int32),
            pltpu.VMEM((RCHUNK, SEQ), jnp.float32),
            pltpu.VMEM((RCHUNK, SEQ), jnp.float32),
            pltpu.VMEM((CTOK,), jnp.int32),
            pltpu.VMEM((CTOK,), jnp.int32),
            pltpu.VMEM((CTOK,), jnp.float32),
            pltpu.VMEM((CTOK,), jnp.float32),
            pltpu.SemaphoreType.DMA,
            pltpu.SemaphoreType.DMA,
            pltpu.SemaphoreType.DMA,
            pltpu.SemaphoreType.DMA,
            pltpu.SemaphoreType.DMA,
            pltpu.SemaphoreType.DMA,
        ],
    )(_gather_kernel)
    return run(idx, table)
